# baseline (speedup 1.0000x reference)
import jax
import jax.numpy as jnp
from jax.experimental import pallas as pl

_MAX_HITS = 32
_FIXED = 128
_MAX_STEPS = 160
_MAX_DEPTH = 10000.0
_BLOCK_R = 128

_DN0 = (((0,), (0,)), ((), ()))


def _cumsum_sub(x, n):
    sh = 1
    while sh < n:
        x = x + jnp.concatenate([jnp.zeros_like(x[:sh]), x[:-sh]], axis=0)
        sh *= 2
    return x


def _body(ro_ref, rdir_ref, vi_ref, tn_ref, tf_ref,
          pts_ref, vout_ref, dout_ref, sout_ref):
    R = tn_ref.shape[0]
    rr = jax.lax.broadcasted_iota(jnp.int32, (R, R), 0)
    rc = jax.lax.broadcasted_iota(jnp.int32, (R, R), 1)
    eye_r = (rr == rc).astype(jnp.float32)
    tn = jax.lax.dot_general(tn_ref[...], eye_r, _DN0,
                             preferred_element_type=jnp.float32)
    tf = jax.lax.dot_general(tf_ref[...], eye_r, _DN0,
                             preferred_element_type=jnp.float32)
    vif = jax.lax.dot_general(vi_ref[...].astype(jnp.float32), eye_r, _DN0,
                              preferred_element_type=jnp.float32)
    vi = vif.astype(jnp.int32)

    rng = jnp.where(vi == -1, 0.0, tf - tn)
    total = jnp.sum(rng, axis=0, keepdims=True)
    prob = rng / total
    cdf = _cumsum_sub(prob, _MAX_HITS)
    pclip = jnp.maximum(prob, 1e-12)
    s = (tf - tn) / pclip
    cdf_prev = jnp.concatenate([jnp.zeros_like(cdf[:1]), cdf[:-1]], axis=0)
    c = tn - cdf_prev * s

    u = (jax.lax.broadcasted_iota(jnp.int32, (_FIXED, R), 0).astype(jnp.float32)
         + 0.5) * (1.0 / _FIXED)
    c_g = jnp.broadcast_to(c[0:1], (_FIXED, R))
    s_g = jnp.broadcast_to(s[0:1], (_FIXED, R))
    v_g = jnp.broadcast_to(vif[0:1], (_FIXED, R))
    for k in range(_MAX_HITS - 1):
        ind = u >= cdf[k:k + 1]
        c_g = jnp.where(ind, c[k + 1:k + 2], c_g)
        s_g = jnp.where(ind, s[k + 1:k + 2], s_g)
        v_g = jnp.where(ind, vif[k + 1:k + 2], v_g)
    t_raw = c_g + s_g * u

    u128 = (_FIXED + 0.5) / _FIXED
    t128 = c[_MAX_HITS - 1:] + s[_MAX_HITS - 1:] * u128
    nxt = jnp.concatenate([t_raw[1:], t128], axis=0)
    prv = jnp.concatenate([t_raw[:1], t_raw[:-1]], axis=0)
    dist = jnp.maximum((nxt - prv) * 0.5, 0.0)

    jj = jax.lax.broadcasted_iota(jnp.int32, (_FIXED, _FIXED), 0)
    cc = jax.lax.broadcasted_iota(jnp.int32, (_FIXED, _FIXED), 1)
    eye = (jj == cc).astype(jnp.float32)
    depth = jax.lax.dot_general(t_raw, eye, _DN0,
                                preferred_element_type=jnp.float32)
    v_out = jax.lax.dot_general(v_g, eye, _DN0,
                                preferred_element_type=jnp.float32)
    dist_out = jax.lax.dot_general(dist, eye, _DN0,
                                   preferred_element_type=jnp.float32)

    tail = _MAX_STEPS - _FIXED
    dout_ref[:, :_FIXED] = depth
    dout_ref[:, _FIXED:] = jnp.full((R, tail), _MAX_DEPTH, jnp.float32)
    vout_ref[:, :_FIXED] = v_out.astype(jnp.int32)
    vout_ref[:, _FIXED:] = jnp.full((R, tail), -1, jnp.int32)
    sout_ref[:, :_FIXED] = dist_out
    sout_ref[:, _FIXED:] = jnp.zeros((R, tail), jnp.float32)

    W = 3 * _MAX_STEPS
    Wh = 3 * _FIXED
    je = jax.lax.broadcasted_iota(jnp.int32, (_FIXED, Wh), 0)
    ie = jax.lax.broadcasted_iota(jnp.int32, (_FIXED, Wh), 1)
    expand = (ie // 3 == je).astype(jnp.float32)
    t_il = jax.lax.dot_general(t_raw, expand, _DN0,
                               preferred_element_type=jnp.float32)
    mod3 = jax.lax.broadcasted_iota(jnp.int32, (3, W), 1) % 3
    ax3 = jax.lax.broadcasted_iota(jnp.int32, (3, W), 0)
    sel3 = (mod3 == ax3).astype(jnp.float32)
    o_il = jnp.dot(ro_ref[...], sel3, preferred_element_type=jnp.float32)
    d_il = jnp.dot(rdir_ref[...], sel3, preferred_element_type=jnp.float32)
    pts_ref[:, :Wh] = o_il[:, :Wh] + t_il * d_il[:, :Wh]
    pts_ref[:, Wh:] = o_il[:, Wh:] + _MAX_DEPTH * d_il[:, Wh:]


def kernel(rays_o, rays_d, vox_idx, t_near, t_far):
    n = rays_o.shape[0]
    grid = (n // _BLOCK_R,)
    row = lambda i: (i, 0)
    col = lambda i: (0, i)
    pts_il, vidx, depth, dists = pl.pallas_call(
        _body,
        grid=grid,
        in_specs=[
            pl.BlockSpec((_BLOCK_R, 3), row),
            pl.BlockSpec((_BLOCK_R, 3), row),
            pl.BlockSpec((_BLOCK_R, _MAX_HITS), row),
            pl.BlockSpec((_BLOCK_R, _MAX_HITS), row),
            pl.BlockSpec((_BLOCK_R, _MAX_HITS), row),
        ],
        out_specs=[
            pl.BlockSpec((_BLOCK_R, 3 * _MAX_STEPS), row),
            pl.BlockSpec((_BLOCK_R, _MAX_STEPS), row),
            pl.BlockSpec((_BLOCK_R, _MAX_STEPS), row),
            pl.BlockSpec((_BLOCK_R, _MAX_STEPS), row),
        ],
        out_shape=[
            jax.ShapeDtypeStruct((n, 3 * _MAX_STEPS), jnp.float32),
            jax.ShapeDtypeStruct((n, _MAX_STEPS), jnp.int32),
            jax.ShapeDtypeStruct((n, _MAX_STEPS), jnp.float32),
            jax.ShapeDtypeStruct((n, _MAX_STEPS), jnp.float32),
        ],
    )(rays_o, rays_d, vox_idx, t_near, t_far)
    pts = pts_il.reshape(n, _MAX_STEPS, 3)
    return (pts, vidx, depth, dists)


# speedup vs baseline: 1.0244x; 1.0244x over previous
import jax
import jax.numpy as jnp
from jax.experimental import pallas as pl

_MAX_HITS = 32
_FIXED = 128
_MAX_STEPS = 160
_MAX_DEPTH = 10000.0
_BLOCK_R = 128

_DN0 = (((0,), (0,)), ((), ()))


def _cumsum_sub(x, n):
    sh = 1
    while sh < n:
        x = x + jnp.concatenate([jnp.zeros_like(x[:sh]), x[:-sh]], axis=0)
        sh *= 2
    return x


def _body(ro_ref, rdir_ref, vi_ref, tn_ref, tf_ref,
          pts_ref, vout_ref, dout_ref, sout_ref):
    R = tn_ref.shape[0]
    tn = jnp.transpose(tn_ref[...])
    tf = jnp.transpose(tf_ref[...])
    vi = jnp.transpose(vi_ref[...])
    vif = vi.astype(jnp.float32)

    rng = jnp.where(vi == -1, 0.0, tf - tn)
    total = jnp.sum(rng, axis=0, keepdims=True)
    prob = rng / total
    cdf = _cumsum_sub(prob, _MAX_HITS)
    pclip = jnp.maximum(prob, 1e-12)
    s = (tf - tn) / pclip
    cdf_prev = jnp.concatenate([jnp.zeros_like(cdf[:1]), cdf[:-1]], axis=0)
    c = tn - cdf_prev * s

    u = (jax.lax.broadcasted_iota(jnp.int32, (_FIXED, R), 0).astype(jnp.float32)
         + 0.5) * (1.0 / _FIXED)
    c_g = jnp.broadcast_to(c[0:1], (_FIXED, R))
    s_g = jnp.broadcast_to(s[0:1], (_FIXED, R))
    v_g = jnp.broadcast_to(vif[0:1], (_FIXED, R))
    for k in range(_MAX_HITS - 1):
        ind = u >= cdf[k:k + 1]
        c_g = jnp.where(ind, c[k + 1:k + 2], c_g)
        s_g = jnp.where(ind, s[k + 1:k + 2], s_g)
        v_g = jnp.where(ind, vif[k + 1:k + 2], v_g)
    t_raw = c_g + s_g * u

    u128 = (_FIXED + 0.5) / _FIXED
    t128 = c[_MAX_HITS - 1:] + s[_MAX_HITS - 1:] * u128
    nxt = jnp.concatenate([t_raw[1:], t128], axis=0)
    prv = jnp.concatenate([t_raw[:1], t_raw[:-1]], axis=0)
    dist = jnp.maximum((nxt - prv) * 0.5, 0.0)

    depth = jnp.transpose(t_raw)
    v_out = jnp.transpose(v_g)
    dist_out = jnp.transpose(dist)

    tail = _MAX_STEPS - _FIXED
    dout_ref[:, :_FIXED] = depth
    dout_ref[:, _FIXED:] = jnp.full((R, tail), _MAX_DEPTH, jnp.float32)
    vout_ref[:, :_FIXED] = v_out.astype(jnp.int32)
    vout_ref[:, _FIXED:] = jnp.full((R, tail), -1, jnp.int32)
    sout_ref[:, :_FIXED] = dist_out
    sout_ref[:, _FIXED:] = jnp.zeros((R, tail), jnp.float32)

    W = 3 * _MAX_STEPS
    Wh = 3 * _FIXED
    je = jax.lax.broadcasted_iota(jnp.int32, (_FIXED, Wh), 0)
    ie = jax.lax.broadcasted_iota(jnp.int32, (_FIXED, Wh), 1)
    expand = (ie // 3 == je).astype(jnp.float32)
    t_il = jnp.dot(depth, expand, preferred_element_type=jnp.float32)
    mod3 = jax.lax.broadcasted_iota(jnp.int32, (3, W), 1) % 3
    ax3 = jax.lax.broadcasted_iota(jnp.int32, (3, W), 0)
    sel3 = (mod3 == ax3).astype(jnp.float32)
    o_il = jnp.dot(ro_ref[...], sel3, preferred_element_type=jnp.float32)
    d_il = jnp.dot(rdir_ref[...], sel3, preferred_element_type=jnp.float32)
    pts_ref[:, :Wh] = o_il[:, :Wh] + t_il * d_il[:, :Wh]
    pts_ref[:, Wh:] = o_il[:, Wh:] + _MAX_DEPTH * d_il[:, Wh:]


def kernel(rays_o, rays_d, vox_idx, t_near, t_far):
    n = rays_o.shape[0]
    grid = (n // _BLOCK_R,)
    row = lambda i: (i, 0)
    col = lambda i: (0, i)
    pts_il, vidx, depth, dists = pl.pallas_call(
        _body,
        grid=grid,
        in_specs=[
            pl.BlockSpec((_BLOCK_R, 3), row),
            pl.BlockSpec((_BLOCK_R, 3), row),
            pl.BlockSpec((_BLOCK_R, _MAX_HITS), row),
            pl.BlockSpec((_BLOCK_R, _MAX_HITS), row),
            pl.BlockSpec((_BLOCK_R, _MAX_HITS), row),
        ],
        out_specs=[
            pl.BlockSpec((_BLOCK_R, 3 * _MAX_STEPS), row),
            pl.BlockSpec((_BLOCK_R, _MAX_STEPS), row),
            pl.BlockSpec((_BLOCK_R, _MAX_STEPS), row),
            pl.BlockSpec((_BLOCK_R, _MAX_STEPS), row),
        ],
        out_shape=[
            jax.ShapeDtypeStruct((n, 3 * _MAX_STEPS), jnp.float32),
            jax.ShapeDtypeStruct((n, _MAX_STEPS), jnp.int32),
            jax.ShapeDtypeStruct((n, _MAX_STEPS), jnp.float32),
            jax.ShapeDtypeStruct((n, _MAX_STEPS), jnp.float32),
        ],
    )(rays_o, rays_d, vox_idx, t_near, t_far)
    pts = pts_il.reshape(n, _MAX_STEPS, 3)
    return (pts, vidx, depth, dists)


# speedup vs baseline: 1.1955x; 1.1671x over previous
import jax
import jax.numpy as jnp
from jax.experimental import pallas as pl

_MAX_HITS = 32
_FIXED = 128
_MAX_STEPS = 160
_MAX_DEPTH = 10000.0
_BLOCK_R = 128

_DN0 = (((0,), (0,)), ((), ()))


def _cumsum_sub(x, n):
    sh = 1
    while sh < n:
        x = x + jnp.concatenate([jnp.zeros_like(x[:sh]), x[:-sh]], axis=0)
        sh *= 2
    return x


def _body(ro_ref, rdir_ref, vi_ref, tn_ref, tf_ref,
          px_ref, py_ref, pz_ref, vout_ref, dout_ref, sout_ref):
    R = tn_ref.shape[0]
    tn = jnp.transpose(tn_ref[...])
    tf = jnp.transpose(tf_ref[...])
    vi = jnp.transpose(vi_ref[...])
    vif = vi.astype(jnp.float32)

    rng = jnp.where(vi == -1, 0.0, tf - tn)
    total = jnp.sum(rng, axis=0, keepdims=True)
    prob = rng / total
    cdf = _cumsum_sub(prob, _MAX_HITS)
    pclip = jnp.maximum(prob, 1e-12)
    s = (tf - tn) / pclip
    cdf_prev = jnp.concatenate([jnp.zeros_like(cdf[:1]), cdf[:-1]], axis=0)
    c = tn - cdf_prev * s

    u = (jax.lax.broadcasted_iota(jnp.int32, (_FIXED, R), 0).astype(jnp.float32)
         + 0.5) * (1.0 / _FIXED)
    c_g = jnp.broadcast_to(c[0:1], (_FIXED, R))
    s_g = jnp.broadcast_to(s[0:1], (_FIXED, R))
    v_g = jnp.broadcast_to(vif[0:1], (_FIXED, R))
    for k in range(_MAX_HITS - 1):
        ind = u >= cdf[k:k + 1]
        c_g = jnp.where(ind, c[k + 1:k + 2], c_g)
        s_g = jnp.where(ind, s[k + 1:k + 2], s_g)
        v_g = jnp.where(ind, vif[k + 1:k + 2], v_g)
    t_raw = c_g + s_g * u

    u128 = (_FIXED + 0.5) / _FIXED
    t128 = c[_MAX_HITS - 1:] + s[_MAX_HITS - 1:] * u128
    nxt = jnp.concatenate([t_raw[1:], t128], axis=0)
    prv = jnp.concatenate([t_raw[:1], t_raw[:-1]], axis=0)
    dist = jnp.maximum((nxt - prv) * 0.5, 0.0)

    depth = jnp.transpose(t_raw)
    v_out = jnp.transpose(v_g)
    dist_out = jnp.transpose(dist)

    tail = _MAX_STEPS - _FIXED
    dout_ref[:, :_FIXED] = depth
    dout_ref[:, _FIXED:] = jnp.full((R, tail), _MAX_DEPTH, jnp.float32)
    vout_ref[:, :_FIXED] = v_out.astype(jnp.int32)
    vout_ref[:, _FIXED:] = jnp.full((R, tail), -1, jnp.int32)
    sout_ref[:, :_FIXED] = dist_out
    sout_ref[:, _FIXED:] = jnp.zeros((R, tail), jnp.float32)

    ro = ro_ref[...]
    rdir = rdir_ref[...]
    depth_full = jnp.concatenate(
        [depth, jnp.full((R, tail), _MAX_DEPTH, jnp.float32)], axis=1)
    px_ref[...] = ro[:, 0:1] + depth_full * rdir[:, 0:1]
    py_ref[...] = ro[:, 1:2] + depth_full * rdir[:, 1:2]
    pz_ref[...] = ro[:, 2:3] + depth_full * rdir[:, 2:3]


def kernel(rays_o, rays_d, vox_idx, t_near, t_far):
    n = rays_o.shape[0]
    grid = (n // _BLOCK_R,)
    row = lambda i: (i, 0)
    col = lambda i: (0, i)
    px, py, pz, vidx, depth, dists = pl.pallas_call(
        _body,
        grid=grid,
        in_specs=[
            pl.BlockSpec((_BLOCK_R, 3), row),
            pl.BlockSpec((_BLOCK_R, 3), row),
            pl.BlockSpec((_BLOCK_R, _MAX_HITS), row),
            pl.BlockSpec((_BLOCK_R, _MAX_HITS), row),
            pl.BlockSpec((_BLOCK_R, _MAX_HITS), row),
        ],
        out_specs=[
            pl.BlockSpec((_BLOCK_R, _MAX_STEPS), row),
            pl.BlockSpec((_BLOCK_R, _MAX_STEPS), row),
            pl.BlockSpec((_BLOCK_R, _MAX_STEPS), row),
            pl.BlockSpec((_BLOCK_R, _MAX_STEPS), row),
            pl.BlockSpec((_BLOCK_R, _MAX_STEPS), row),
            pl.BlockSpec((_BLOCK_R, _MAX_STEPS), row),
        ],
        out_shape=[
            jax.ShapeDtypeStruct((n, _MAX_STEPS), jnp.float32),
            jax.ShapeDtypeStruct((n, _MAX_STEPS), jnp.float32),
            jax.ShapeDtypeStruct((n, _MAX_STEPS), jnp.float32),
            jax.ShapeDtypeStruct((n, _MAX_STEPS), jnp.int32),
            jax.ShapeDtypeStruct((n, _MAX_STEPS), jnp.float32),
            jax.ShapeDtypeStruct((n, _MAX_STEPS), jnp.float32),
        ],
    )(rays_o, rays_d, vox_idx, t_near, t_far)
    pts = jnp.stack([px, py, pz], axis=-1)
    return (pts, vidx, depth, dists)


# speedup vs baseline: 5.5755x; 4.6636x over previous
import jax
import jax.numpy as jnp
from jax.experimental import pallas as pl

_MAX_HITS = 32
_FIXED = 128
_MAX_STEPS = 160
_MAX_DEPTH = 10000.0
_BLOCK_R = 1024
_CHUNK_R = 128


def _cumsum_sub(x, n):
    sh = 1
    while sh < n:
        x = x + jnp.concatenate([jnp.zeros_like(x[:sh]), x[:-sh]], axis=0)
        sh *= 2
    return x


def _body(ro_ref, rd_ref, vi_ref, tn_ref, tf_ref,
          pts_ref, vout_ref, dout_ref, sout_ref):
    tail = _MAX_STEPS - _FIXED
    for s0 in range(0, _BLOCK_R, _CHUNK_R):
        R = _CHUNK_R
        sl = pl.ds(s0, R)
        tn = tn_ref[:, sl]
        tf = tf_ref[:, sl]
        vi = vi_ref[:, sl]
        vif = vi.astype(jnp.float32)

        rng = jnp.where(vi == -1, 0.0, tf - tn)
        total = jnp.sum(rng, axis=0, keepdims=True)
        prob = rng / total
        cdf = _cumsum_sub(prob, _MAX_HITS)
        pclip = jnp.maximum(prob, 1e-12)
        s = (tf - tn) / pclip
        cdf_prev = jnp.concatenate([jnp.zeros_like(cdf[:1]), cdf[:-1]], axis=0)
        c = tn - cdf_prev * s

        u = (jax.lax.broadcasted_iota(jnp.int32, (_FIXED, R), 0)
             .astype(jnp.float32) + 0.5) * (1.0 / _FIXED)
        c_g = jnp.broadcast_to(c[0:1], (_FIXED, R))
        s_g = jnp.broadcast_to(s[0:1], (_FIXED, R))
        v_g = jnp.broadcast_to(vif[0:1], (_FIXED, R))
        for k in range(_MAX_HITS - 1):
            ind = u >= cdf[k:k + 1]
            c_g = jnp.where(ind, c[k + 1:k + 2], c_g)
            s_g = jnp.where(ind, s[k + 1:k + 2], s_g)
            v_g = jnp.where(ind, vif[k + 1:k + 2], v_g)
        t_raw = c_g + s_g * u

        u128 = (_FIXED + 0.5) / _FIXED
        t128 = c[_MAX_HITS - 1:] + s[_MAX_HITS - 1:] * u128
        nxt = jnp.concatenate([t_raw[1:], t128], axis=0)
        prv = jnp.concatenate([t_raw[:1], t_raw[:-1]], axis=0)
        dist = jnp.maximum((nxt - prv) * 0.5, 0.0)

        dout_ref[:_FIXED, sl] = t_raw
        dout_ref[_FIXED:, sl] = jnp.full((tail, R), _MAX_DEPTH, jnp.float32)
        vout_ref[:_FIXED, sl] = v_g.astype(jnp.int32)
        vout_ref[_FIXED:, sl] = jnp.full((tail, R), -1, jnp.int32)
        sout_ref[:_FIXED, sl] = dist
        sout_ref[_FIXED:, sl] = jnp.zeros((tail, R), jnp.float32)

        ro = ro_ref[:, sl]
        rd = rd_ref[:, sl]
        for ax in range(3):
            o_row = ro[ax:ax + 1]
            d_row = rd[ax:ax + 1]
            base = ax * _MAX_STEPS
            pts_ref[base:base + _FIXED, sl] = o_row + t_raw * d_row
            pts_ref[base + _FIXED:base + _MAX_STEPS, sl] = jnp.broadcast_to(
                o_row + _MAX_DEPTH * d_row, (tail, R))


def kernel(rays_o, rays_d, vox_idx, t_near, t_far):
    n = rays_o.shape[0]
    grid = (n // _BLOCK_R,)
    col = lambda i: (0, i)
    pts3, vidx_t, depth_t, dists_t = pl.pallas_call(
        _body,
        grid=grid,
        in_specs=[
            pl.BlockSpec((3, _BLOCK_R), col),
            pl.BlockSpec((3, _BLOCK_R), col),
            pl.BlockSpec((_MAX_HITS, _BLOCK_R), col),
            pl.BlockSpec((_MAX_HITS, _BLOCK_R), col),
            pl.BlockSpec((_MAX_HITS, _BLOCK_R), col),
        ],
        out_specs=[
            pl.BlockSpec((3 * _MAX_STEPS, _BLOCK_R), col),
            pl.BlockSpec((_MAX_STEPS, _BLOCK_R), col),
            pl.BlockSpec((_MAX_STEPS, _BLOCK_R), col),
            pl.BlockSpec((_MAX_STEPS, _BLOCK_R), col),
        ],
        out_shape=[
            jax.ShapeDtypeStruct((3 * _MAX_STEPS, n), jnp.float32),
            jax.ShapeDtypeStruct((_MAX_STEPS, n), jnp.int32),
            jax.ShapeDtypeStruct((_MAX_STEPS, n), jnp.float32),
            jax.ShapeDtypeStruct((_MAX_STEPS, n), jnp.float32),
        ],
    )(rays_o.T, rays_d.T, vox_idx.T, t_near.T, t_far.T)
    pts = jnp.transpose(pts3.reshape(3, _MAX_STEPS, n), (2, 1, 0))
    return (pts, vidx_t.T, depth_t.T, dists_t.T)
